# baseline (speedup 1.0000x reference)
import functools

import jax
import jax.numpy as jnp
from jax import lax
from jax.experimental import pallas as pl
from jax.experimental.pallas import tpu as pltpu
from jax.experimental.pallas import tpu_sc as plsc

_NC = 2
_NS = 16
_NW = _NC * _NS


@functools.lru_cache(maxsize=None)
def _make_gather(V, D, NB, NF, interpret=False):
    UB = 1024
    NBLK = NB // UB
    NU = NF * NBLK
    assert NU % _NW == 0, NU
    upw = NU // _NW
    mesh = plsc.VectorSubcoreMesh(core_axis_name="c", subcore_axis_name="s")

    @functools.partial(
        pl.kernel,
        out_type=jax.ShapeDtypeStruct((NF, D, NB), jnp.float32),
        mesh=mesh,
        scratch_types=[
            pltpu.VMEM((UB,), jnp.int32),
            pltpu.VMEM((D, UB), jnp.float32),
            pltpu.SemaphoreType.DMA,
        ],
        compiler_params=pltpu.CompilerParams(use_tc_tiling_on_sc=False),
        interpret=interpret,
    )
    def gather_kernel(tT_hbm, xT_hbm, out_hbm, idx_v, buf_v, sem):
        wid = lax.axis_index("s") * _NC + lax.axis_index("c")

        @pl.loop(0, upw)
        def _unit(j):
            u = wid * upw + j
            f = u // NBLK
            b0 = (u % NBLK) * UB
            pltpu.sync_copy(xT_hbm.at[f, pl.ds(b0, UB)], idx_v)
            descs = [
                pltpu.async_copy(tT_hbm.at[d].at[idx_v], buf_v.at[d], sem)
                for d in range(D)
            ]
            for desc in descs:
                desc.wait()
            pltpu.sync_copy(buf_v, out_hbm.at[f].at[:, pl.ds(b0, UB)])

    return gather_kernel


def kernel(x, table):
    V, D = table.shape
    NB, NF = x.shape
    outT = _make_gather(V, D, NB, NF)(table.T, x.T)
    return outT.transpose(2, 0, 1)


# speedup vs baseline: 2.5784x; 2.5784x over previous
import functools

import jax
import jax.numpy as jnp
from jax import lax
from jax.experimental import pallas as pl
from jax.experimental.pallas import tpu as pltpu
from jax.experimental.pallas import tpu_sc as plsc

_NC = 2
_NS = 16
_NW = _NC * _NS


@functools.lru_cache(maxsize=None)
def _make_gather(V, D, NB, NF, interpret=False):
    UB = 1024
    NBLK = NB // UB
    NU = NF * NBLK
    assert NU % _NW == 0, NU
    upw = NU // _NW
    mesh = plsc.VectorSubcoreMesh(core_axis_name="c", subcore_axis_name="s")

    @functools.partial(
        pl.kernel,
        out_type=jax.ShapeDtypeStruct((NF, D, NB), jnp.float32),
        mesh=mesh,
        scratch_types=[
            pltpu.VMEM((UB,), jnp.int32),
            pltpu.VMEM((UB, D), jnp.float32),
            pltpu.VMEM((D, UB), jnp.float32),
            pltpu.SemaphoreType.DMA,
        ],
        compiler_params=pltpu.CompilerParams(
            use_tc_tiling_on_sc=False, needs_layout_passes=False
        ),
        interpret=interpret,
    )
    def gather_kernel(t_hbm, xT_hbm, out_hbm, idx_v, rows_v, dbuf_v, sem):
        wid = lax.axis_index("s") * _NC + lax.axis_index("c")
        lane = lax.iota(jnp.int32, 16)

        @pl.loop(0, upw)
        def _unit(j):
            u = wid * upw + j
            f = u // NBLK
            b0 = (u % NBLK) * UB
            pltpu.sync_copy(xT_hbm.at[f, pl.ds(b0, UB)], idx_v)
            pltpu.async_copy(t_hbm.at[idx_v], rows_v, sem).wait()

            @pl.loop(0, UB // 16)
            def _grp(g):
                idx_b = g * 16 + lane
                for d in range(D):
                    vec = plsc.load_gather(
                        rows_v, [idx_b, jnp.full((16,), d, jnp.int32)]
                    )
                    dbuf_v[d, pl.ds(g * 16, 16)] = vec

            pltpu.sync_copy(dbuf_v, out_hbm.at[f].at[:, pl.ds(b0, UB)])

    return gather_kernel


def kernel(x, table):
    V, D = table.shape
    NB, NF = x.shape
    outT = _make_gather(V, D, NB, NF)(table, x.T)
    return outT.transpose(2, 0, 1)


# speedup vs baseline: 2.6888x; 1.0428x over previous
import functools

import jax
import jax.numpy as jnp
from jax import lax
from jax.experimental import pallas as pl
from jax.experimental.pallas import tpu as pltpu
from jax.experimental.pallas import tpu_sc as plsc

_NC = 2
_NS = 16
_NW = _NC * _NS


@functools.lru_cache(maxsize=None)
def _make_gather(V, D, NB, NF, interpret=False):
    UB = 1024
    NBLK = NB // UB
    NU = NF * NBLK
    assert NU % _NW == 0, NU
    upw = NU // _NW
    mesh = plsc.VectorSubcoreMesh(core_axis_name="c", subcore_axis_name="s")

    @functools.partial(
        pl.kernel,
        out_type=jax.ShapeDtypeStruct((NF, D, NB), jnp.float32),
        mesh=mesh,
        scratch_types=[
            pltpu.VMEM((2, UB), jnp.int32),
            pltpu.VMEM((2, UB, D), jnp.float32),
            pltpu.VMEM((2, D, UB), jnp.float32),
            pltpu.SemaphoreType.DMA,
            pltpu.SemaphoreType.DMA,
            pltpu.SemaphoreType.DMA,
            pltpu.SemaphoreType.DMA,
        ],
        compiler_params=pltpu.CompilerParams(
            use_tc_tiling_on_sc=False,
            needs_layout_passes=False,
            disable_bounds_checks=True,
        ),
        interpret=interpret,
    )
    def gather_kernel(t_hbm, xT_hbm, out_hbm, idx_v, rows_v, dbuf_v,
                      gsem0, gsem1, osem0, osem1):
        wid = lax.axis_index("s") * _NC + lax.axis_index("c")
        u0 = wid * upw
        lane = lax.iota(jnp.int32, 16)
        gsems = (gsem0, gsem1)
        osems = (osem0, osem1)

        def stage_and_fire(j):
            b = j % 2
            u = u0 + j
            f = u // NBLK
            b0 = (u % NBLK) * UB
            pltpu.sync_copy(xT_hbm.at[f, pl.ds(b0, UB)], idx_v.at[b])
            return pltpu.async_copy(
                t_hbm.at[idx_v.at[b]], rows_v.at[b], gsems[b]
            )

        def transpose(j):
            b = j % 2

            @pl.loop(0, UB // 16, unroll=2)
            def _grp(g):
                idx_b = g * 16 + lane
                for d in range(D):
                    vec = plsc.load_gather(
                        rows_v.at[b], [idx_b, jnp.full((16,), d, jnp.int32)]
                    )
                    dbuf_v[b, d, pl.ds(g * 16, 16)] = vec

        def fire_out(j):
            b = j % 2
            u = u0 + j
            f = u // NBLK
            b0 = (u % NBLK) * UB
            return pltpu.async_copy(
                dbuf_v.at[b], out_hbm.at[f].at[:, pl.ds(b0, UB)], osems[b]
            )

        gdescs = [None, None]
        odescs = [None, None]
        gdescs[0] = stage_and_fire(0)
        for j in range(upw):
            b = j % 2
            if j + 1 < upw:
                gdescs[(j + 1) % 2] = stage_and_fire(j + 1)
            gdescs[b].wait()
            if odescs[b] is not None:
                odescs[b].wait()
            transpose(j)
            odescs[b] = fire_out(j)
        odescs[0].wait()
        odescs[1].wait()

    return gather_kernel


def kernel(x, table):
    V, D = table.shape
    NB, NF = x.shape
    outT = _make_gather(V, D, NB, NF)(table, x.T)
    return outT.transpose(2, 0, 1)


# speedup vs baseline: 3.0118x; 1.1201x over previous
import functools

import jax
import jax.numpy as jnp
from jax import lax
from jax.experimental import pallas as pl
from jax.experimental.pallas import tpu as pltpu
from jax.experimental.pallas import tpu_sc as plsc

_NC = 2
_NS = 16
_NW = _NC * _NS


@functools.lru_cache(maxsize=None)
def _make_gather(V, D, NB, NF, interpret=False):
    UB = 1024
    NBLK = NB // UB
    NU = NF * NBLK
    assert NU % _NW == 0, NU
    upw = NU // _NW
    mesh = plsc.VectorSubcoreMesh(core_axis_name="c", subcore_axis_name="s")

    @functools.partial(
        pl.kernel,
        out_type=jax.ShapeDtypeStruct((NF, D, NB), jnp.float32),
        mesh=mesh,
        scratch_types=[
            pltpu.VMEM((2, UB), jnp.int32),
            pltpu.VMEM((2, UB, D), jnp.float32),
            pltpu.VMEM((2, D, UB), jnp.float32),
            pltpu.SemaphoreType.DMA,
            pltpu.SemaphoreType.DMA,
            pltpu.SemaphoreType.DMA,
            pltpu.SemaphoreType.DMA,
        ],
        compiler_params=pltpu.CompilerParams(
            use_tc_tiling_on_sc=False,
            needs_layout_passes=False,
            disable_bounds_checks=True,
        ),
        interpret=interpret,
    )
    def gather_kernel(t_hbm, xT_hbm, out_hbm, idx_v, rows_v, dbuf_v,
                      gsem0, gsem1, osem0, osem1):
        wid = lax.axis_index("s") * _NC + lax.axis_index("c")
        u0 = wid * upw
        lane = lax.iota(jnp.int32, 16)
        gsems = (gsem0, gsem1)
        osems = (osem0, osem1)

        def stage_and_fire(j):
            b = j % 2
            u = u0 + j
            f = u // NBLK
            b0 = (u % NBLK) * UB
            pltpu.sync_copy(xT_hbm.at[f, pl.ds(b0, UB)], idx_v.at[b])
            return pltpu.async_copy(
                t_hbm.at[idx_v.at[b]], rows_v.at[b], gsems[b]
            )

        def transpose(j):
            b = j % 2
            dsplat = [jnp.full((16,), d, jnp.int32) for d in range(D)]

            @plsc.parallel_loop(0, UB // 16, unroll=2)
            def _grp(g):
                idx_b = g * 16 + lane
                for d in range(D):
                    vec = plsc.load_gather(rows_v.at[b], [idx_b, dsplat[d]])
                    dbuf_v[b, d, pl.ds(g * 16, 16)] = vec

        def fire_out(j):
            b = j % 2
            u = u0 + j
            f = u // NBLK
            b0 = (u % NBLK) * UB
            return pltpu.async_copy(
                dbuf_v.at[b], out_hbm.at[f].at[:, pl.ds(b0, UB)], osems[b]
            )

        gdescs = [None, None]
        odescs = [None, None]
        gdescs[0] = stage_and_fire(0)
        for j in range(upw):
            b = j % 2
            if j + 1 < upw:
                gdescs[(j + 1) % 2] = stage_and_fire(j + 1)
            gdescs[b].wait()
            if odescs[b] is not None:
                odescs[b].wait()
            transpose(j)
            odescs[b] = fire_out(j)
        odescs[0].wait()
        odescs[1].wait()

    return gather_kernel


def kernel(x, table):
    V, D = table.shape
    NB, NF = x.shape
    outT = _make_gather(V, D, NB, NF)(table, x.T)
    return outT.transpose(2, 0, 1)
